# baseline (speedup 1.0000x reference)
import functools

import jax
import jax.numpy as jnp
from jax import lax
from jax.experimental import pallas as pl
from jax.experimental.pallas import tpu as pltpu
from jax.experimental.pallas import tpu_sc as plsc

_H, _W = 496, 432
_HW = _H * _W
_B, _P, _C = 4, 12000, 64
_NBANDS = 8
_NR = _H // _NBANDS
_CH = _NR * _W
_K = 1200
_NK = _P // _K
_VK = _K // 16
_NV = _CH // 16
_NC, _NS = 2, 16

_mesh = plsc.VectorSubcoreMesh(
    core_axis_name="c", subcore_axis_name="s", num_cores=_NC, num_subcores=_NS
)


@functools.partial(
    pl.kernel,
    out_type=jax.ShapeDtypeStruct((_B * _C * _HW,), jnp.float32),
    mesh=_mesh,
    compiler_params=pltpu.CompilerParams(needs_layout_passes=False),
    scratch_types=[
        pltpu.VMEM((4 * _K,), jnp.int32),
        pltpu.VMEM((4 * _K,), jnp.int32),
        pltpu.VMEM((_CH,), jnp.int32),
        pltpu.VMEM((_P + 16,), jnp.int32),
        pltpu.VMEM((_P,), jnp.float32),
        pltpu.VMEM((_P,), jnp.float32),
        pltpu.VMEM((_CH + 16,), jnp.float32),
        pltpu.VMEM((_CH + 16,), jnp.float32),
        pltpu.SemaphoreType.DMA,
        pltpu.SemaphoreType.DMA,
        pltpu.SemaphoreType.DMA,
        pltpu.SemaphoreType.DMA,
        pltpu.SemaphoreType.DMA,
        pltpu.SemaphoreType.DMA,
    ],
)
def _pseudo_image_kernel(
    coords_hbm, ft_hbm, out_hbm,
    cb0, cb1, mapv, listv, t0, t1, o0, o1,
    sc0, sc1, st0, st1, so0, so1,
):
    wid = lax.axis_index("s") * _NC + lax.axis_index("c")
    b = wid // _NBANDS
    y0 = (wid % _NBANDS) * _NR

    cbufs, csems = (cb0, cb1), (sc0, sc1)
    tbufs, tsems = (t0, t1), (st0, st1)
    obufs, osems = (o0, o1), (so0, so1)
    i16 = lax.iota(jnp.int32, 16)

    def in_copy(k, par):
        off = pl.multiple_of((b * _P + k * _K) * 4, 8)
        return pltpu.make_async_copy(
            coords_hbm.at[pl.ds(off, 4 * _K)], cbufs[par], csems[par]
        )

    in_copy(0, 0).start()
    in_copy(1, 1).start()

    empty = jnp.full((16,), _P, dtype=jnp.int32)

    @plsc.parallel_loop(0, _NV, unroll=6)
    def _init_body(v):
        mapv[pl.ds(v * 16, 16)] = empty

    zero16 = jnp.zeros((16,), jnp.float32)

    @plsc.parallel_loop(0, (_CH + 16) // 16, unroll=6)
    def _z0(v):
        o0[pl.ds(v * 16, 16)] = zero16

    @plsc.parallel_loop(0, (_CH + 16) // 16, unroll=6)
    def _z1(v):
        o1[pl.ds(v * 16, 16)] = zero16

    def do_chunk(k, par):
        in_copy(k, par).wait()
        cb = cbufs[par]
        base = k * _K

        def chunk_body(v, _):
            i4 = (v * 16 + i16) * 4
            yv = plsc.load_gather(cb, [i4 + 1])
            xv = plsc.load_gather(cb, [i4 + 2])
            valid = (xv >= 0) & (xv < _W) & (yv >= y0) & (yv < y0 + _NR)
            flat = (yv - y0) * _W + xv
            p = base + v * 16 + i16
            plsc.store_scatter(mapv, [flat], p, mask=valid)
            for _r in range(2):
                rb = plsc.load_gather(mapv, [flat], mask=valid)
                m2 = valid & (p > rb)
                plsc.store_scatter(mapv, [flat], p, mask=m2)
            return 0

        lax.fori_loop(0, _VK, chunk_body, 0)

    def p1_body(i, _):
        do_chunk(2 * i, 0)

        @pl.when(i < _NK // 2 - 1)
        def _():
            in_copy(2 * i + 2, 0).start()

        do_chunk(2 * i + 1, 1)

        @pl.when(i < _NK // 2 - 1)
        def _():
            in_copy(2 * i + 3, 1).start()

        return 0

    lax.fori_loop(0, _NK // 2, p1_body, 0)

    @plsc.parallel_loop(0, _NV, unroll=2, carry=jnp.int32(0))
    def cnt(v, n):
        m = mapv[pl.ds(v * 16, 16)]
        keep = m != _P
        w = ((v * 16 + i16) << 14) | m
        plsc.store_compressed(listv.at[pl.ds(n, 16)], w, mask=keep)
        return n + jnp.sum(keep.astype(jnp.int32))

    listv[pl.ds(cnt, 16)] = jnp.full((16,), _CH << 14, dtype=jnp.int32)
    ngroups = (cnt + 15) // 16

    def tab_copy(c, par):
        off = pl.multiple_of((b * _C + c) * _P, 8)
        return pltpu.make_async_copy(
            ft_hbm.at[pl.ds(off, _P)], tbufs[par], tsems[par]
        )

    def out_copy(c, par):
        off = pl.multiple_of((b * _C + c) * _HW + y0 * _W, 8)
        return pltpu.make_async_copy(
            obufs[par].at[pl.ds(0, _CH)], out_hbm.at[pl.ds(off, _CH)], osems[par]
        )

    tab_copy(0, 0).start()
    tab_copy(1, 1).start()

    def do_channel(j, c, par):
        tb, ob = tbufs[par], obufs[par]
        tab_copy(c, par).wait()

        @pl.when(j > 0)
        def _():
            out_copy(c, par).wait()

        @plsc.parallel_loop(0, ngroups, unroll=4)
        def _val_body(g):
            w = listv[pl.ds(g * 16, 16)]
            cell = lax.shift_right_logical(w, 14)
            p = w & 0x3FFF
            plsc.store_scatter(ob, [cell], plsc.load_gather(tb, [p]))

        out_copy(c, par).start()

        @pl.when(j < _C // 2 - 1)
        def _():
            tab_copy(c + 2, par).start()

    def p2_body(j, _):
        do_channel(j, 2 * j, 0)
        do_channel(j, 2 * j + 1, 1)
        return 0

    lax.fori_loop(0, _C // 2, p2_body, 0)

    out_copy(_C - 2, 0).wait()
    out_copy(_C - 1, 1).wait()


def _tr_body(x_ref, o_ref):
    o_ref[...] = jnp.transpose(x_ref[...], (0, 2, 1))


_transpose = pl.pallas_call(
    _tr_body,
    grid=(_B,),
    in_specs=[pl.BlockSpec((1, _P, _C), lambda b: (b, 0, 0))],
    out_specs=pl.BlockSpec((1, _C, _P), lambda b: (b, 0, 0)),
    out_shape=jax.ShapeDtypeStruct((_B, _C, _P), jnp.float32),
)


def kernel(pillar_features, coords):
    ft = _transpose(pillar_features.astype(jnp.float32))
    out = _pseudo_image_kernel(
        coords.astype(jnp.int32).reshape(-1), ft.reshape(-1)
    )
    return out.reshape(_B, _C, _H, _W)


# speedup vs baseline: 5.5860x; 5.5860x over previous
import functools

import jax
import jax.numpy as jnp
from jax import lax
from jax.experimental import pallas as pl
from jax.experimental.pallas import tpu as pltpu
from jax.experimental.pallas import tpu_sc as plsc

_H, _W = 496, 432
_B, _P, _C = 4, 12000, 64
_XB = 56
_XB7 = _W - 7 * _XB
_CH = _XB * _H
_NV = _CH // 16
_K = 1200
_NK = _P // _K
_VK = _K // 16
_NC, _NS = 2, 16

_mesh = plsc.VectorSubcoreMesh(
    core_axis_name="c", subcore_axis_name="s", num_cores=_NC, num_subcores=_NS
)


@functools.partial(
    pl.kernel,
    out_type=jax.ShapeDtypeStruct((_B, _C, _W, _H), jnp.float32),
    mesh=_mesh,
    compiler_params=pltpu.CompilerParams(needs_layout_passes=False),
    scratch_types=[
        pltpu.VMEM((4 * _K,), jnp.int32),
        pltpu.VMEM((4 * _K,), jnp.int32),
        pltpu.VMEM((_CH,), jnp.int32),
        pltpu.VMEM((_P + 16,), jnp.int32),
        pltpu.VMEM((_P,), jnp.float32),
        pltpu.VMEM((_P,), jnp.float32),
        pltpu.VMEM((_XB, _H), jnp.float32),
        pltpu.VMEM((_XB, _H), jnp.float32),
        pltpu.SemaphoreType.DMA,
        pltpu.SemaphoreType.DMA,
        pltpu.SemaphoreType.DMA,
        pltpu.SemaphoreType.DMA,
        pltpu.SemaphoreType.DMA,
        pltpu.SemaphoreType.DMA,
    ],
)
def _pseudo_image_kernel(
    coords_hbm, ft_hbm, out_hbm,
    cb0, cb1, mapv, listv, t0, t1, o0, o1,
    sc0, sc1, st0, st1, so0, so1,
):
    wid = lax.axis_index("s") * _NC + lax.axis_index("c")
    b = wid // 8
    r = wid % 8
    x0 = r * _XB
    xhi = x0 + jnp.where(r == 7, _XB7, _XB)

    cbufs, csems = (cb0, cb1), (sc0, sc1)
    tbufs, tsems = (t0, t1), (st0, st1)
    obufs, osems = (o0, o1), (so0, so1)
    i16 = lax.iota(jnp.int32, 16)

    def in_copy(k, par):
        off = pl.multiple_of((b * _P + k * _K) * 4, 8)
        return pltpu.make_async_copy(
            coords_hbm.at[pl.ds(off, 4 * _K)], cbufs[par], csems[par]
        )

    in_copy(0, 0).start()
    in_copy(1, 1).start()

    empty = jnp.full((16,), _P, dtype=jnp.int32)

    @plsc.parallel_loop(0, _NV, unroll=6)
    def _init_body(v):
        mapv[pl.ds(v * 16, 16)] = empty

    zero16 = jnp.zeros((16,), jnp.float32)
    nzrow = _H // 16

    @plsc.parallel_loop(0, _XB * nzrow, unroll=4)
    def _z0(t):
        o0[t // nzrow, pl.ds((t % nzrow) * 16, 16)] = zero16

    @plsc.parallel_loop(0, _XB * nzrow, unroll=4)
    def _z1(t):
        o1[t // nzrow, pl.ds((t % nzrow) * 16, 16)] = zero16

    def do_chunk(k, par):
        in_copy(k, par).wait()
        cb = cbufs[par]
        base = k * _K

        def chunk_body(v, _):
            i4 = (v * 16 + i16) * 4
            yv = plsc.load_gather(cb, [i4 + 1])
            xv = plsc.load_gather(cb, [i4 + 2])
            valid = (yv >= 0) & (yv < _H) & (xv >= x0) & (xv < xhi)
            flat = (xv - x0) * _H + yv
            p = base + v * 16 + i16
            plsc.store_scatter(mapv, [flat], p, mask=valid)
            for _r in range(2):
                rb = plsc.load_gather(mapv, [flat], mask=valid)
                m2 = valid & (p > rb)
                plsc.store_scatter(mapv, [flat], p, mask=m2)
            return 0

        lax.fori_loop(0, _VK, chunk_body, 0)

    def p1_body(i, _):
        do_chunk(2 * i, 0)

        @pl.when(i < _NK // 2 - 1)
        def _():
            in_copy(2 * i + 2, 0).start()

        do_chunk(2 * i + 1, 1)

        @pl.when(i < _NK // 2 - 1)
        def _():
            in_copy(2 * i + 3, 1).start()

        return 0

    lax.fori_loop(0, _NK // 2, p1_body, 0)

    @plsc.parallel_loop(0, _NV, unroll=2, carry=jnp.int32(0))
    def cnt(v, n):
        m = mapv[pl.ds(v * 16, 16)]
        keep = m != _P
        cell = v * 16 + i16
        w = ((cell // _H) << 23) | ((cell % _H) << 14) | m
        plsc.store_compressed(listv.at[pl.ds(n, 16)], w, mask=keep)
        return n + jnp.sum(keep.astype(jnp.int32))

    ngroups = (cnt + 15) // 16

    def tab_copy(c, par):
        off = pl.multiple_of((b * _C + c) * _P, 8)
        return pltpu.make_async_copy(
            ft_hbm.at[pl.ds(off, _P)], tbufs[par], tsems[par]
        )

    def out_copy_full(c, par):
        return pltpu.make_async_copy(
            obufs[par].at[pl.ds(0, _XB), :],
            out_hbm.at[b, c, pl.ds(pl.multiple_of(x0, 8), _XB), :],
            osems[par],
        )

    def out_copy_last(c, par):
        return pltpu.make_async_copy(
            obufs[par].at[pl.ds(0, _XB7), :],
            out_hbm.at[b, c, pl.ds(pl.multiple_of(x0, 8), _XB7), :],
            osems[par],
        )

    def out_start(c, par):
        @pl.when(r < 7)
        def _():
            out_copy_full(c, par).start()

        @pl.when(r == 7)
        def _():
            out_copy_last(c, par).start()

    def out_wait(c, par):
        @pl.when(r < 7)
        def _():
            out_copy_full(c, par).wait()

        @pl.when(r == 7)
        def _():
            out_copy_last(c, par).wait()

    tab_copy(0, 0).start()
    tab_copy(1, 1).start()

    def do_channel(j, c, par):
        tb, ob = tbufs[par], obufs[par]
        tab_copy(c, par).wait()

        @pl.when(j > 0)
        def _():
            out_wait(c, par)

        @plsc.parallel_loop(0, ngroups, unroll=4)
        def _val_body(g):
            live = g * 16 + i16 < cnt
            w = listv[pl.ds(g * 16, 16)]
            xr = lax.shift_right_logical(w, 23)
            yc = lax.shift_right_logical(w, 14) & 0x1FF
            p = w & 0x3FFF
            vals = plsc.load_gather(tb, [p], mask=live)
            plsc.store_scatter(ob, [xr, yc], vals, mask=live)

        out_start(c, par)

        @pl.when(j < _C // 2 - 1)
        def _():
            tab_copy(c + 2, par).start()

    def p2_body(j, _):
        do_channel(j, 2 * j, 0)
        do_channel(j, 2 * j + 1, 1)
        return 0

    lax.fori_loop(0, _C // 2, p2_body, 0)

    out_wait(_C - 2, 0)
    out_wait(_C - 1, 1)


def _tr_body(x_ref, o_ref):
    o_ref[...] = jnp.transpose(x_ref[...], (0, 2, 1))


_transpose = pl.pallas_call(
    _tr_body,
    grid=(_B,),
    in_specs=[pl.BlockSpec((1, _P, _C), lambda b: (b, 0, 0))],
    out_specs=pl.BlockSpec((1, _C, _P), lambda b: (b, 0, 0)),
    out_shape=jax.ShapeDtypeStruct((_B, _C, _P), jnp.float32),
)


def kernel(pillar_features, coords):
    ft = _transpose(pillar_features.astype(jnp.float32))
    out = _pseudo_image_kernel(
        coords.astype(jnp.int32).reshape(-1), ft.reshape(-1)
    )
    return jnp.transpose(out, (0, 1, 3, 2))


# speedup vs baseline: 5.8310x; 1.0438x over previous
import functools

import jax
import jax.numpy as jnp
from jax import lax
from jax.experimental import pallas as pl
from jax.experimental.pallas import tpu as pltpu
from jax.experimental.pallas import tpu_sc as plsc

_H, _W = 496, 432
_B, _P, _C = 4, 12000, 64
_XB = 56
_XB7 = _W - 7 * _XB
_CH = _XB * _H
_NV = _CH // 16
_PS = 12288
_K = 1200
_NK = _P // _K
_VK = _K // 16
_NC, _NS = 2, 16

_mesh = plsc.VectorSubcoreMesh(
    core_axis_name="c", subcore_axis_name="s", num_cores=_NC, num_subcores=_NS
)


@functools.partial(
    pl.kernel,
    out_type=jax.ShapeDtypeStruct((_B, _C, _W, _H), jnp.float32),
    mesh=_mesh,
    compiler_params=pltpu.CompilerParams(needs_layout_passes=False),
    scratch_types=[
        pltpu.VMEM((4 * _K,), jnp.int32),
        pltpu.VMEM((4 * _K,), jnp.int32),
        pltpu.VMEM((_CH,), jnp.int32),
        pltpu.VMEM((_P + 16,), jnp.int32),
        pltpu.VMEM((_P,), jnp.float32),
        pltpu.VMEM((_P,), jnp.float32),
        pltpu.VMEM((_XB, _H), jnp.float32),
        pltpu.VMEM((_XB, _H), jnp.float32),
        pltpu.SemaphoreType.DMA,
        pltpu.SemaphoreType.DMA,
        pltpu.SemaphoreType.DMA,
        pltpu.SemaphoreType.DMA,
        pltpu.SemaphoreType.DMA,
        pltpu.SemaphoreType.DMA,
    ],
)
def _pseudo_image_kernel(
    coords_hbm, ft_hbm, out_hbm,
    cb0, cb1, mapv, listv, t0, t1, o0, o1,
    sc0, sc1, st0, st1, so0, so1,
):
    wid = lax.axis_index("s") * _NC + lax.axis_index("c")
    b = wid // 8
    r = wid % 8
    x0 = r * _XB
    xhi = x0 + jnp.where(r == 7, _XB7, _XB)

    cbufs, csems = (cb0, cb1), (sc0, sc1)
    tbufs, tsems = (t0, t1), (st0, st1)
    obufs, osems = (o0, o1), (so0, so1)
    i16 = lax.iota(jnp.int32, 16)

    def in_copy(k, par):
        off = pl.multiple_of((b * _P + k * _K) * 4, 8)
        return pltpu.make_async_copy(
            coords_hbm.at[pl.ds(off, 4 * _K)], cbufs[par], csems[par]
        )

    in_copy(0, 0).start()
    in_copy(1, 1).start()

    empty = jnp.full((16,), _P, dtype=jnp.int32)

    @plsc.parallel_loop(0, _NV, unroll=6)
    def _init_body(v):
        mapv[pl.ds(v * 16, 16)] = empty

    zero16 = jnp.zeros((16,), jnp.float32)
    nzrow = _H // 16

    @plsc.parallel_loop(0, _XB * nzrow, unroll=4)
    def _z0(t):
        o0[t // nzrow, pl.ds((t % nzrow) * 16, 16)] = zero16

    @plsc.parallel_loop(0, _XB * nzrow, unroll=4)
    def _z1(t):
        o1[t // nzrow, pl.ds((t % nzrow) * 16, 16)] = zero16

    def do_chunk(k, par):
        in_copy(k, par).wait()
        cb = cbufs[par]
        base = k * _K

        def chunk_body(v, _):
            i4 = (v * 16 + i16) * 4
            yv = plsc.load_gather(cb, [i4 + 1])
            xv = plsc.load_gather(cb, [i4 + 2])
            valid = (yv >= 0) & (yv < _H) & (xv >= x0) & (xv < xhi)
            flat = (xv - x0) * _H + yv
            p = base + v * 16 + i16
            plsc.store_scatter(mapv, [flat], p, mask=valid)
            for _r in range(2):
                rb = plsc.load_gather(mapv, [flat], mask=valid)
                m2 = valid & (p > rb)
                plsc.store_scatter(mapv, [flat], p, mask=m2)
            return 0

        lax.fori_loop(0, _VK, chunk_body, 0)

    def p1_body(i, _):
        do_chunk(2 * i, 0)

        @pl.when(i < _NK // 2 - 1)
        def _():
            in_copy(2 * i + 2, 0).start()

        do_chunk(2 * i + 1, 1)

        @pl.when(i < _NK // 2 - 1)
        def _():
            in_copy(2 * i + 3, 1).start()

        return 0

    lax.fori_loop(0, _NK // 2, p1_body, 0)

    @plsc.parallel_loop(0, _NV, unroll=2, carry=jnp.int32(0))
    def cnt(v, n):
        m = mapv[pl.ds(v * 16, 16)]
        keep = m != _P
        cell = v * 16 + i16
        w = ((cell // _H) << 23) | ((cell % _H) << 14) | m
        plsc.store_compressed(listv.at[pl.ds(n, 16)], w, mask=keep)
        return n + jnp.sum(keep.astype(jnp.int32))

    ngroups = (cnt + 15) // 16

    def tab_copy(c, par):
        off = pl.multiple_of((b * _C + c) * _PS, 8)
        return pltpu.make_async_copy(
            ft_hbm.at[pl.ds(off, _P)], tbufs[par], tsems[par]
        )

    def out_copy_full(c, par):
        return pltpu.make_async_copy(
            obufs[par].at[pl.ds(0, _XB), :],
            out_hbm.at[b, c, pl.ds(pl.multiple_of(x0, 8), _XB), :],
            osems[par],
        )

    def out_copy_last(c, par):
        return pltpu.make_async_copy(
            obufs[par].at[pl.ds(0, _XB7), :],
            out_hbm.at[b, c, pl.ds(pl.multiple_of(x0, 8), _XB7), :],
            osems[par],
        )

    def out_start(c, par):
        @pl.when(r < 7)
        def _():
            out_copy_full(c, par).start()

        @pl.when(r == 7)
        def _():
            out_copy_last(c, par).start()

    def out_wait(c, par):
        @pl.when(r < 7)
        def _():
            out_copy_full(c, par).wait()

        @pl.when(r == 7)
        def _():
            out_copy_last(c, par).wait()

    tab_copy(0, 0).start()
    tab_copy(1, 1).start()

    def do_channel(j, c, par):
        tb, ob = tbufs[par], obufs[par]
        tab_copy(c, par).wait()

        @pl.when(j > 0)
        def _():
            out_wait(c, par)

        @plsc.parallel_loop(0, ngroups, unroll=4)
        def _val_body(g):
            live = g * 16 + i16 < cnt
            w = listv[pl.ds(g * 16, 16)]
            xr = lax.shift_right_logical(w, 23)
            yc = lax.shift_right_logical(w, 14) & 0x1FF
            p = w & 0x3FFF
            vals = plsc.load_gather(tb, [p], mask=live)
            plsc.store_scatter(ob, [xr, yc], vals, mask=live)

        out_start(c, par)

        @pl.when(j < _C // 2 - 1)
        def _():
            tab_copy(c + 2, par).start()

    def p2_body(j, _):
        do_channel(j, 2 * j, 0)
        do_channel(j, 2 * j + 1, 1)
        return 0

    lax.fori_loop(0, _C // 2, p2_body, 0)

    out_wait(_C - 2, 0)
    out_wait(_C - 1, 1)


def _tr_body(x_ref, o_ref):
    xt = jnp.transpose(x_ref[0], (1, 0))
    for c in range(_C):
        o_ref[pl.ds(c * _PS, _P)] = xt[c]


_transpose = pl.pallas_call(
    _tr_body,
    grid=(_B,),
    in_specs=[pl.BlockSpec((1, _P, _C), lambda b: (b, 0, 0))],
    out_specs=pl.BlockSpec((_C * _PS,), lambda b: (b,)),
    out_shape=jax.ShapeDtypeStruct((_B * _C * _PS,), jnp.float32),
)


def kernel(pillar_features, coords):
    ft = _transpose(pillar_features.astype(jnp.float32))
    out = _pseudo_image_kernel(coords.astype(jnp.int32).reshape(-1), ft)
    return jnp.transpose(out, (0, 1, 3, 2))


# speedup vs baseline: 6.4011x; 1.0978x over previous
import functools

import jax
import jax.numpy as jnp
from jax import lax
from jax.experimental import pallas as pl
from jax.experimental.pallas import tpu as pltpu
from jax.experimental.pallas import tpu_sc as plsc

_H, _W = 496, 432
_B, _P, _C = 4, 12000, 64
_XB = 56
_XB7 = _W - 7 * _XB
_CH = _XB * _H
_NV = _CH // 16
_PS = 12288
_K = 1200
_NK = _P // _K
_VK = _K // 16
_NC, _NS = 2, 16

_mesh = plsc.VectorSubcoreMesh(
    core_axis_name="c", subcore_axis_name="s", num_cores=_NC, num_subcores=_NS
)


@functools.partial(
    pl.kernel,
    out_type=jax.ShapeDtypeStruct((_B, _C, _W, _H), jnp.float32),
    mesh=_mesh,
    compiler_params=pltpu.CompilerParams(needs_layout_passes=False),
    scratch_types=[
        pltpu.VMEM((4 * _K,), jnp.int32),
        pltpu.VMEM((4 * _K,), jnp.int32),
        pltpu.VMEM((_CH,), jnp.int32),
        pltpu.VMEM((_P + 16,), jnp.int32),
        pltpu.VMEM((_P,), jnp.float32),
        pltpu.VMEM((_P,), jnp.float32),
        pltpu.VMEM((_XB, _H), jnp.float32),
        pltpu.VMEM((_XB, _H), jnp.float32),
        pltpu.SemaphoreType.DMA,
        pltpu.SemaphoreType.DMA,
        pltpu.SemaphoreType.DMA,
        pltpu.SemaphoreType.DMA,
        pltpu.SemaphoreType.DMA,
        pltpu.SemaphoreType.DMA,
    ],
)
def _pseudo_image_kernel(
    coords_hbm, ft_hbm, out_hbm,
    cb0, cb1, mapv, listv, t0, t1, o0, o1,
    sc0, sc1, st0, st1, so0, so1,
):
    wid = lax.axis_index("s") * _NC + lax.axis_index("c")
    b = wid // 8
    r = wid % 8
    x0 = r * _XB
    xhi = x0 + jnp.where(r == 7, _XB7, _XB)

    cbufs, csems = (cb0, cb1), (sc0, sc1)
    tbufs, tsems = (t0, t1), (st0, st1)
    obufs, osems = (o0, o1), (so0, so1)
    i16 = lax.iota(jnp.int32, 16)

    def in_copy(k, par):
        off = pl.multiple_of((b * _P + k * _K) * 4, 8)
        return pltpu.make_async_copy(
            coords_hbm.at[pl.ds(off, 4 * _K)], cbufs[par], csems[par]
        )

    in_copy(0, 0).start()
    in_copy(1, 1).start()

    empty = jnp.full((16,), _P, dtype=jnp.int32)

    @plsc.parallel_loop(0, _NV, unroll=6)
    def _init_body(v):
        mapv[pl.ds(v * 16, 16)] = empty

    zero16 = jnp.zeros((16,), jnp.float32)
    nzrow = _H // 16

    @plsc.parallel_loop(0, _XB * nzrow, unroll=4)
    def _z0(t):
        o0[t // nzrow, pl.ds((t % nzrow) * 16, 16)] = zero16

    @plsc.parallel_loop(0, _XB * nzrow, unroll=4)
    def _z1(t):
        o1[t // nzrow, pl.ds((t % nzrow) * 16, 16)] = zero16

    def do_chunk(k, par):
        in_copy(k, par).wait()
        cb = cbufs[par]
        base = k * _K

        def chunk_body(v, _):
            i4 = (v * 16 + i16) * 4
            yv = plsc.load_gather(cb, [i4 + 1])
            xv = plsc.load_gather(cb, [i4 + 2])
            valid = (yv >= 0) & (yv < _H) & (xv >= x0) & (xv < xhi)
            flat = (xv - x0) * _H + yv
            p = base + v * 16 + i16
            plsc.store_scatter(mapv, [flat], p, mask=valid)
            for _r in range(2):
                rb = plsc.load_gather(mapv, [flat], mask=valid)
                m2 = valid & (p > rb)
                plsc.store_scatter(mapv, [flat], p, mask=m2)
            return 0

        lax.fori_loop(0, _VK, chunk_body, 0)

    def p1_body(i, _):
        do_chunk(2 * i, 0)

        @pl.when(i < _NK // 2 - 1)
        def _():
            in_copy(2 * i + 2, 0).start()

        do_chunk(2 * i + 1, 1)

        @pl.when(i < _NK // 2 - 1)
        def _():
            in_copy(2 * i + 3, 1).start()

        return 0

    lax.fori_loop(0, _NK // 2, p1_body, 0)

    @plsc.parallel_loop(0, _NV, unroll=2, carry=jnp.int32(0))
    def cnt(v, n):
        m = mapv[pl.ds(v * 16, 16)]
        keep = m != _P
        cell = v * 16 + i16
        w = ((cell // _H) << 23) | ((cell % _H) << 14) | m
        plsc.store_compressed(listv.at[pl.ds(n, 16)], w, mask=keep)
        return n + jnp.sum(keep.astype(jnp.int32))

    ngroups = (cnt + 15) // 16

    def tab_copy(c, par):
        off = pl.multiple_of((b * _C + c) * _PS, 8)
        return pltpu.make_async_copy(
            ft_hbm.at[pl.ds(off, _P)], tbufs[par], tsems[par]
        )

    def out_copy_full(c, par):
        return pltpu.make_async_copy(
            obufs[par].at[pl.ds(0, _XB), :],
            out_hbm.at[b, c, pl.ds(pl.multiple_of(x0, 8), _XB), :],
            osems[par],
        )

    def out_copy_last(c, par):
        return pltpu.make_async_copy(
            obufs[par].at[pl.ds(0, _XB7), :],
            out_hbm.at[b, c, pl.ds(pl.multiple_of(x0, 8), _XB7), :],
            osems[par],
        )

    def out_start(c, par):
        @pl.when(r < 7)
        def _():
            out_copy_full(c, par).start()

        @pl.when(r == 7)
        def _():
            out_copy_last(c, par).start()

    def out_wait(c, par):
        @pl.when(r < 7)
        def _():
            out_copy_full(c, par).wait()

        @pl.when(r == 7)
        def _():
            out_copy_last(c, par).wait()

    tab_copy(0, 0).start()
    tab_copy(1, 1).start()

    def do_channel(j, c, par):
        tb, ob = tbufs[par], obufs[par]
        tab_copy(c, par).wait()

        @pl.when(j > 0)
        def _():
            out_wait(c, par)

        @plsc.parallel_loop(0, ngroups, unroll=4)
        def _val_body(g):
            live = g * 16 + i16 < cnt
            w = listv[pl.ds(g * 16, 16)]
            xr = lax.shift_right_logical(w, 23)
            yc = lax.shift_right_logical(w, 14) & 0x1FF
            p = w & 0x3FFF
            vals = plsc.load_gather(tb, [p], mask=live)
            plsc.store_scatter(ob, [xr, yc], vals, mask=live)

        out_start(c, par)

        @pl.when(j < _C // 2 - 1)
        def _():
            tab_copy(c + 2, par).start()

    def p2_body(j, _):
        do_channel(j, 2 * j, 0)
        do_channel(j, 2 * j + 1, 1)
        return 0

    lax.fori_loop(0, _C // 2, p2_body, 0)

    out_wait(_C - 2, 0)
    out_wait(_C - 1, 1)


def _stage_body(x_ref, o_ref):
    for c in range(_C):
        o_ref[pl.ds(c * _PS, _P)] = x_ref[0, c, :]


_stage = pl.pallas_call(
    _stage_body,
    grid=(_B,),
    in_specs=[pl.BlockSpec((1, _C, _P), lambda b: (b, 0, 0))],
    out_specs=pl.BlockSpec((_C * _PS,), lambda b: (b,)),
    out_shape=jax.ShapeDtypeStruct((_B * _C * _PS,), jnp.float32),
)


def kernel(pillar_features, coords):
    ft = _stage(jnp.transpose(pillar_features.astype(jnp.float32), (0, 2, 1)))
    out = _pseudo_image_kernel(coords.astype(jnp.int32).reshape(-1), ft)
    return jnp.transpose(out, (0, 1, 3, 2))


# speedup vs baseline: 7.5870x; 1.1853x over previous
import functools

import jax
import jax.numpy as jnp
from jax import lax
from jax.experimental import pallas as pl
from jax.experimental.pallas import tpu as pltpu
from jax.experimental.pallas import tpu_sc as plsc

_H, _W = 496, 432
_B, _P, _C = 4, 12000, 64
_XB = 56
_XB7 = _W - 7 * _XB
_CH = _XB * _H
_NV = _CH // 16
_PS = 12288
_K = 1200
_NK = _P // _K
_VK = _K // 16
_NC, _NS = 2, 16

_mesh = plsc.VectorSubcoreMesh(
    core_axis_name="c", subcore_axis_name="s", num_cores=_NC, num_subcores=_NS
)


@functools.partial(
    pl.kernel,
    out_type=jax.ShapeDtypeStruct((_B, _C, _W, _H), jnp.float32),
    mesh=_mesh,
    compiler_params=pltpu.CompilerParams(needs_layout_passes=False),
    scratch_types=[
        pltpu.VMEM((_K,), jnp.int32),
        pltpu.VMEM((_K,), jnp.int32),
        pltpu.VMEM((_K,), jnp.int32),
        pltpu.VMEM((_K,), jnp.int32),
        pltpu.VMEM((_CH,), jnp.int32),
        pltpu.VMEM((_P + 16,), jnp.int32),
        pltpu.VMEM((_P,), jnp.float32),
        pltpu.VMEM((_P,), jnp.float32),
        pltpu.VMEM((_XB, _H), jnp.float32),
        pltpu.VMEM((_XB, _H), jnp.float32),
        pltpu.SemaphoreType.DMA,
        pltpu.SemaphoreType.DMA,
        pltpu.SemaphoreType.DMA,
        pltpu.SemaphoreType.DMA,
        pltpu.SemaphoreType.DMA,
        pltpu.SemaphoreType.DMA,
    ],
)
def _pseudo_image_kernel(
    cyx_hbm, ft_hbm, out_hbm,
    yb0, yb1, xb0, xb1, mapv, listv, t0, t1, o0, o1,
    sc0, sc1, st0, st1, so0, so1,
):
    wid = lax.axis_index("s") * _NC + lax.axis_index("c")
    b = wid // 8
    r = wid % 8
    x0 = r * _XB
    xhi = x0 + jnp.where(r == 7, _XB7, _XB)

    ybufs, xbufs, csems = (yb0, yb1), (xb0, xb1), (sc0, sc1)
    tbufs, tsems = (t0, t1), (st0, st1)
    obufs, osems = (o0, o1), (so0, so1)
    i16 = lax.iota(jnp.int32, 16)

    def in_copies(k, par):
        offy = pl.multiple_of((b * 4 + 1) * _PS + k * _K, 8)
        offx = pl.multiple_of((b * 4 + 2) * _PS + k * _K, 8)
        return (
            pltpu.make_async_copy(
                cyx_hbm.at[pl.ds(offy, _K)], ybufs[par], csems[par]
            ),
            pltpu.make_async_copy(
                cyx_hbm.at[pl.ds(offx, _K)], xbufs[par], csems[par]
            ),
        )

    def in_start(k, par):
        cy, cx = in_copies(k, par)
        cy.start()
        cx.start()

    in_start(0, 0)
    in_start(1, 1)

    empty = jnp.full((16,), _P, dtype=jnp.int32)

    @plsc.parallel_loop(0, _NV, unroll=6)
    def _init_body(v):
        mapv[pl.ds(v * 16, 16)] = empty

    zero16 = jnp.zeros((16,), jnp.float32)
    nzrow = _H // 16

    @plsc.parallel_loop(0, _XB * nzrow, unroll=4)
    def _z0(t):
        o0[t // nzrow, pl.ds((t % nzrow) * 16, 16)] = zero16

    @plsc.parallel_loop(0, _XB * nzrow, unroll=4)
    def _z1(t):
        o1[t // nzrow, pl.ds((t % nzrow) * 16, 16)] = zero16

    def do_chunk(k, par):
        cy, cx = in_copies(k, par)
        cy.wait()
        cx.wait()
        yb, xb = ybufs[par], xbufs[par]
        base = k * _K

        def chunk_body(v, _):
            yv = yb[pl.ds(v * 16, 16)]
            xv = xb[pl.ds(v * 16, 16)]
            valid = (yv >= 0) & (yv < _H) & (xv >= x0) & (xv < xhi)
            flat = (xv - x0) * _H + yv
            p = base + v * 16 + i16
            plsc.store_scatter(mapv, [flat], p, mask=valid)
            for _r in range(2):
                rb = plsc.load_gather(mapv, [flat], mask=valid)
                m2 = valid & (p > rb)
                plsc.store_scatter(mapv, [flat], p, mask=m2)
            return 0

        lax.fori_loop(0, _VK, chunk_body, 0)

    def p1_body(i, _):
        do_chunk(2 * i, 0)

        @pl.when(i < _NK // 2 - 1)
        def _():
            in_start(2 * i + 2, 0)

        do_chunk(2 * i + 1, 1)

        @pl.when(i < _NK // 2 - 1)
        def _():
            in_start(2 * i + 3, 1)

        return 0

    lax.fori_loop(0, _NK // 2, p1_body, 0)

    @plsc.parallel_loop(0, _NV, unroll=2, carry=jnp.int32(0))
    def cnt(v, n):
        m = mapv[pl.ds(v * 16, 16)]
        keep = m != _P
        cell = v * 16 + i16
        w = ((cell // _H) << 23) | ((cell % _H) << 14) | m
        plsc.store_compressed(listv.at[pl.ds(n, 16)], w, mask=keep)
        return n + jnp.sum(keep.astype(jnp.int32))

    ngroups = (cnt + 15) // 16

    def tab_copy(c, par):
        off = pl.multiple_of((b * _C + c) * _PS, 8)
        return pltpu.make_async_copy(
            ft_hbm.at[pl.ds(off, _P)], tbufs[par], tsems[par]
        )

    def out_copy_full(c, par):
        return pltpu.make_async_copy(
            obufs[par].at[pl.ds(0, _XB), :],
            out_hbm.at[b, c, pl.ds(pl.multiple_of(x0, 8), _XB), :],
            osems[par],
        )

    def out_copy_last(c, par):
        return pltpu.make_async_copy(
            obufs[par].at[pl.ds(0, _XB7), :],
            out_hbm.at[b, c, pl.ds(pl.multiple_of(x0, 8), _XB7), :],
            osems[par],
        )

    def out_start(c, par):
        @pl.when(r < 7)
        def _():
            out_copy_full(c, par).start()

        @pl.when(r == 7)
        def _():
            out_copy_last(c, par).start()

    def out_wait(c, par):
        @pl.when(r < 7)
        def _():
            out_copy_full(c, par).wait()

        @pl.when(r == 7)
        def _():
            out_copy_last(c, par).wait()

    tab_copy(0, 0).start()
    tab_copy(1, 1).start()

    def do_channel(j, c, par):
        tb, ob = tbufs[par], obufs[par]
        tab_copy(c, par).wait()

        @pl.when(j > 0)
        def _():
            out_wait(c, par)

        @plsc.parallel_loop(0, ngroups, unroll=4)
        def _val_body(g):
            live = g * 16 + i16 < cnt
            w = listv[pl.ds(g * 16, 16)]
            xr = lax.shift_right_logical(w, 23)
            yc = lax.shift_right_logical(w, 14) & 0x1FF
            p = w & 0x3FFF
            vals = plsc.load_gather(tb, [p], mask=live)
            plsc.store_scatter(ob, [xr, yc], vals, mask=live)

        out_start(c, par)

        @pl.when(j < _C // 2 - 1)
        def _():
            tab_copy(c + 2, par).start()

    def p2_body(j, _):
        do_channel(j, 2 * j, 0)
        do_channel(j, 2 * j + 1, 1)
        return 0

    lax.fori_loop(0, _C // 2, p2_body, 0)

    out_wait(_C - 2, 0)
    out_wait(_C - 1, 1)


def _stage_body(x_ref, c_ref, o_ref, cyx_ref):
    for c in range(_C):
        o_ref[pl.ds(c * _PS, _P)] = x_ref[0, c, :]
    for col in range(4):
        cyx_ref[pl.ds(col * _PS, _P)] = c_ref[0, col, :]


_stage = pl.pallas_call(
    _stage_body,
    grid=(_B,),
    in_specs=[
        pl.BlockSpec((1, _C, _P), lambda b: (b, 0, 0)),
        pl.BlockSpec((1, 4, _P), lambda b: (b, 0, 0)),
    ],
    out_specs=[
        pl.BlockSpec((_C * _PS,), lambda b: (b,)),
        pl.BlockSpec((4 * _PS,), lambda b: (b,)),
    ],
    out_shape=[
        jax.ShapeDtypeStruct((_B * _C * _PS,), jnp.float32),
        jax.ShapeDtypeStruct((_B * 4 * _PS,), jnp.int32),
    ],
)


def kernel(pillar_features, coords):
    ft, cyx = _stage(
        jnp.transpose(pillar_features.astype(jnp.float32), (0, 2, 1)),
        jnp.transpose(coords.astype(jnp.int32), (0, 2, 1)),
    )
    out = _pseudo_image_kernel(cyx, ft)
    return jnp.transpose(out, (0, 1, 3, 2))
